# baseline (speedup 1.0000x reference)
import functools

import jax
import jax.numpy as jnp
from jax import lax
from jax.experimental import pallas as pl
from jax.experimental.pallas import tpu as pltpu
from jax.experimental.pallas import tpu_sc as plsc

B = 128
N = 32768
TOPK = 4
NC = 2
NS = 16
NW = NC * NS
ROWS_PER_W = B // NW
L = 16
NSETS = 4
NCHUNK = 8
CH = N // NCHUNK

_mesh = plsc.VectorSubcoreMesh(core_axis_name="c", subcore_axis_name="s")


def _insert4(t, v):
    t1, t2, t3, t4 = t
    m1 = jnp.maximum(t1, v)
    y = jnp.minimum(t1, v)
    m2 = jnp.maximum(t2, y)
    y = jnp.minimum(t2, y)
    m3 = jnp.maximum(t3, y)
    y = jnp.minimum(t3, y)
    m4 = jnp.maximum(t4, y)
    return (m1, m2, m3, m4)


@functools.partial(
    pl.kernel,
    out_type=jax.ShapeDtypeStruct((B, N), jnp.float32),
    mesh=_mesh,
    scratch_types=[
        pltpu.VMEM((N,), jnp.float32),
        pltpu.VMEM((N,), jnp.float32),
        pltpu.VMEM((N,), jnp.float32),
        pltpu.SemaphoreType.DMA,
        pltpu.SemaphoreType.DMA,
        pltpu.SemaphoreType.DMA,
        pltpu.SemaphoreType.DMA,
    ],
)
def _wta_sc(x_hbm, r_hbm, out_hbm, xa, xb, rbuf, sem_x, sem_r0, sem_r1,
            sem_out):
    wid = lax.axis_index("s") * NC + lax.axis_index("c")
    neg = jnp.float32(-jnp.inf)

    dnums = lax.GatherDimensionNumbers(
        offset_dims=(), collapsed_slice_dims=(0,), start_index_map=(0,))

    def shuffle(v, idx):
        return lax.gather(v, idx.reshape(L, 1), dnums, slice_sizes=(1,),
                          mode=lax.GatherScatterMode.PROMISE_IN_BOUNDS)

    lane = lax.iota(jnp.int32, L)

    def allmax(v):
        for s in (8, 4, 2, 1):
            v = jnp.maximum(v, shuffle(v, lane ^ s))
        return v

    base = wid * ROWS_PER_W
    xbufs = [xa, xb]
    rsems = [sem_r0, sem_r1]

    h_x = pltpu.async_copy(x_hbm.at[base], xa, sem_x)
    h_out = None

    for row in range(ROWS_PER_W):
        ridx = base + row
        cur = xbufs[row % 2]
        nxt = xbufs[(row + 1) % 2]

        h_x.wait()

        h_r = pltpu.async_copy(
            r_hbm.at[ridx, pl.ds(0, CH)], rbuf.at[pl.ds(0, CH)], rsems[0])
        zero4 = (jnp.full((L,), neg, jnp.float32),) * TOPK
        sets = (zero4,) * NSETS
        for c in range(NCHUNK):
            h_r.wait()
            if c + 1 < NCHUNK:
                h_r = pltpu.async_copy(
                    r_hbm.at[ridx, pl.ds((c + 1) * CH, CH)],
                    rbuf.at[pl.ds((c + 1) * CH, CH)],
                    rsems[(c + 1) % 2])

            def p1_body(i, s, _c=c):
                out = []
                for u in range(NSETS):
                    off = _c * CH + i + u * L
                    v = cur[pl.ds(off, L)] * rbuf[pl.ds(off, L)]
                    out.append(_insert4(s[u], v))
                return tuple(out)

            sets = plsc.parallel_loop(0, CH, step=NSETS * L, unroll=8,
                                      carry=sets)(p1_body)

        if row + 1 < ROWS_PER_W:
            if h_out is not None:
                h_out.wait()
                h_out = None
            h_x = pltpu.async_copy(x_hbm.at[ridx + 1], nxt, sem_x)

        merged = sets[0]
        for u in range(1, NSETS):
            for v in sets[u]:
                merged = _insert4(merged, v)
        t1, t2, t3, t4 = merged

        thr = jnp.full((L,), neg, jnp.float32)
        for i in range(TOPK):
            m = allmax(jnp.maximum(jnp.maximum(t1, t2),
                                   jnp.maximum(t3, t4)))
            if i < TOPK - 1:
                t1 = jnp.where(t1 >= m, neg, t1)
                t2 = jnp.where(t2 >= m, neg, t2)
                t3 = jnp.where(t3 >= m, neg, t3)
                t4 = jnp.where(t4 >= m, neg, t4)
            thr = m

        def p2_body(i):
            xv = cur[pl.ds(i, L)]
            v = xv * rbuf[pl.ds(i, L)]
            cur[pl.ds(i, L)] = jnp.where(v >= thr, xv, jnp.float32(0.0))

        plsc.parallel_loop(0, N, step=L, unroll=16)(p2_body)

        h_out_new = pltpu.async_copy(cur, out_hbm.at[ridx], sem_out)
        if h_out is not None:
            h_out.wait()
        h_out = h_out_new

    h_out.wait()


import numpy as _np


def _threefry_uniform(seed, shape):
    rot_a = (13, 15, 26, 6)
    rot_b = (17, 29, 16, 24)
    k1 = _np.uint32(_np.int64(seed) >> 32)
    k2 = _np.uint32(_np.int64(seed) & 0xFFFFFFFF)
    ks = [k1, k2, _np.uint32(k1 ^ k2 ^ _np.uint32(0x1BD11BDA))]
    size = int(_np.prod(shape))
    idx = _np.arange(size, dtype=_np.uint64)
    x0 = (idx >> _np.uint64(32)).astype(_np.uint32)
    x1 = (idx & _np.uint64(0xFFFFFFFF)).astype(_np.uint32)
    x0 = (x0 + ks[0]).astype(_np.uint32)
    x1 = (x1 + ks[1]).astype(_np.uint32)
    for g in range(5):
        for r in rot_a if g % 2 == 0 else rot_b:
            x0 = (x0 + x1).astype(_np.uint32)
            x1 = (x1 << _np.uint32(r)) | (x1 >> _np.uint32(32 - r))
            x1 = x1 ^ x0
        x0 = (x0 + ks[(g + 1) % 3]).astype(_np.uint32)
        x1 = (x1 + ks[(g + 2) % 3] + _np.uint32(g + 1)).astype(_np.uint32)
    bits = x0 ^ x1
    f = (bits >> _np.uint32(9)) | _np.uint32(0x3F800000)
    f = f.view(_np.float32) - _np.float32(1.0)
    return _np.maximum(_np.float32(0.0), f).reshape(shape)


_R_NP = _threefry_uniform(42, (B, N))


def kernel(x):
    return _wta_sc(x, _R_NP)


# speedup vs baseline: 1.1240x; 1.1240x over previous
import functools

import jax
import jax.numpy as jnp
from jax import lax
from jax.experimental import pallas as pl
from jax.experimental.pallas import tpu as pltpu
from jax.experimental.pallas import tpu_sc as plsc

B = 128
N = 32768
TOPK = 4
NC = 2
NS = 16
NW = NC * NS
ROWS_PER_W = B // NW
L = 16
NSETS = 4
NCHUNK = 4
CH = N // NCHUNK

_mesh = plsc.VectorSubcoreMesh(core_axis_name="c", subcore_axis_name="s")


def _insert4(t, v):
    t1, t2, t3, t4 = t
    m1 = jnp.maximum(t1, v)
    y = jnp.minimum(t1, v)
    m2 = jnp.maximum(t2, y)
    y = jnp.minimum(t2, y)
    m3 = jnp.maximum(t3, y)
    y = jnp.minimum(t3, y)
    m4 = jnp.maximum(t4, y)
    return (m1, m2, m3, m4)


@functools.partial(
    pl.kernel,
    out_type=jax.ShapeDtypeStruct((B, N), jnp.float32),
    mesh=_mesh,
    scratch_types=[
        pltpu.VMEM((N,), jnp.float32),
        pltpu.VMEM((N,), jnp.float32),
        pltpu.VMEM((N,), jnp.float32),
        pltpu.SemaphoreType.DMA,
        pltpu.SemaphoreType.DMA,
        pltpu.SemaphoreType.DMA,
        pltpu.SemaphoreType.DMA,
    ],
)
def _wta_sc(x_hbm, r_hbm, out_hbm, xa, xb, rbuf, sem_x, sem_r0, sem_r1,
            sem_out):
    wid = lax.axis_index("s") * NC + lax.axis_index("c")
    neg = jnp.float32(-jnp.inf)

    dnums = lax.GatherDimensionNumbers(
        offset_dims=(), collapsed_slice_dims=(0,), start_index_map=(0,))

    def shuffle(v, idx):
        return lax.gather(v, idx.reshape(L, 1), dnums, slice_sizes=(1,),
                          mode=lax.GatherScatterMode.PROMISE_IN_BOUNDS)

    lane = lax.iota(jnp.int32, L)

    def allmax(v):
        for s in (8, 4, 2, 1):
            v = jnp.maximum(v, shuffle(v, lane ^ s))
        return v

    base = wid * ROWS_PER_W
    xbufs = [xa, xb]
    rsems = [sem_r0, sem_r1]

    h_x = pltpu.async_copy(x_hbm.at[base], xa, sem_x)
    h_out = None

    for row in range(ROWS_PER_W):
        ridx = base + row
        cur = xbufs[row % 2]
        nxt = xbufs[(row + 1) % 2]

        h_x.wait()

        h_r = pltpu.async_copy(
            r_hbm.at[ridx, pl.ds(0, CH)], rbuf.at[pl.ds(0, CH)], rsems[0])
        zero4 = (jnp.full((L,), neg, jnp.float32),) * TOPK
        sets = (zero4,) * NSETS
        for c in range(NCHUNK):
            h_r.wait()
            if c + 1 < NCHUNK:
                h_r = pltpu.async_copy(
                    r_hbm.at[ridx, pl.ds((c + 1) * CH, CH)],
                    rbuf.at[pl.ds((c + 1) * CH, CH)],
                    rsems[(c + 1) % 2])

            def p1_body(i, s, _c=c):
                out = []
                for u in range(NSETS):
                    off = _c * CH + i + u * L
                    v = cur[pl.ds(off, L)] * rbuf[pl.ds(off, L)]
                    out.append(_insert4(s[u], v))
                return tuple(out)

            sets = plsc.parallel_loop(0, CH, step=NSETS * L, unroll=2,
                                      carry=sets)(p1_body)

        if row + 1 < ROWS_PER_W:
            if h_out is not None:
                h_out.wait()
                h_out = None
            h_x = pltpu.async_copy(x_hbm.at[ridx + 1], nxt, sem_x)

        merged = sets[0]
        for u in range(1, NSETS):
            for v in sets[u]:
                merged = _insert4(merged, v)
        t1, t2, t3, t4 = merged

        thr = jnp.full((L,), neg, jnp.float32)
        for i in range(TOPK):
            m = allmax(jnp.maximum(jnp.maximum(t1, t2),
                                   jnp.maximum(t3, t4)))
            if i < TOPK - 1:
                t1 = jnp.where(t1 >= m, neg, t1)
                t2 = jnp.where(t2 >= m, neg, t2)
                t3 = jnp.where(t3 >= m, neg, t3)
                t4 = jnp.where(t4 >= m, neg, t4)
            thr = m

        def p2_body(i):
            xv = cur[pl.ds(i, L)]
            v = xv * rbuf[pl.ds(i, L)]
            cur[pl.ds(i, L)] = jnp.where(v >= thr, xv, jnp.float32(0.0))

        plsc.parallel_loop(0, N, step=L, unroll=4)(p2_body)

        h_out_new = pltpu.async_copy(cur, out_hbm.at[ridx], sem_out)
        if h_out is not None:
            h_out.wait()
        h_out = h_out_new

    h_out.wait()


import numpy as _np


def _threefry_uniform(seed, shape):
    rot_a = (13, 15, 26, 6)
    rot_b = (17, 29, 16, 24)
    k1 = _np.uint32(_np.int64(seed) >> 32)
    k2 = _np.uint32(_np.int64(seed) & 0xFFFFFFFF)
    ks = [k1, k2, _np.uint32(k1 ^ k2 ^ _np.uint32(0x1BD11BDA))]
    size = int(_np.prod(shape))
    idx = _np.arange(size, dtype=_np.uint64)
    x0 = (idx >> _np.uint64(32)).astype(_np.uint32)
    x1 = (idx & _np.uint64(0xFFFFFFFF)).astype(_np.uint32)
    x0 = (x0 + ks[0]).astype(_np.uint32)
    x1 = (x1 + ks[1]).astype(_np.uint32)
    for g in range(5):
        for r in rot_a if g % 2 == 0 else rot_b:
            x0 = (x0 + x1).astype(_np.uint32)
            x1 = (x1 << _np.uint32(r)) | (x1 >> _np.uint32(32 - r))
            x1 = x1 ^ x0
        x0 = (x0 + ks[(g + 1) % 3]).astype(_np.uint32)
        x1 = (x1 + ks[(g + 2) % 3] + _np.uint32(g + 1)).astype(_np.uint32)
    bits = x0 ^ x1
    f = (bits >> _np.uint32(9)) | _np.uint32(0x3F800000)
    f = f.view(_np.float32) - _np.float32(1.0)
    return _np.maximum(_np.float32(0.0), f).reshape(shape)


_R_NP = _threefry_uniform(42, (B, N))


def kernel(x):
    return _wta_sc(x, _R_NP)


# speedup vs baseline: 1.1795x; 1.0494x over previous
import functools

import jax
import jax.numpy as jnp
from jax import lax
from jax.experimental import pallas as pl
from jax.experimental.pallas import tpu as pltpu
from jax.experimental.pallas import tpu_sc as plsc

B = 128
N = 32768
TOPK = 4
NC = 2
NS = 16
NW = NC * NS
ROWS_PER_W = B // NW
L = 16
NSETS = 4
NCHUNK = 4
CH = N // NCHUNK

_mesh = plsc.VectorSubcoreMesh(core_axis_name="c", subcore_axis_name="s")


def _insert4(t, v):
    t1, t2, t3, t4 = t
    m1 = jnp.maximum(t1, v)
    y = jnp.minimum(t1, v)
    m2 = jnp.maximum(t2, y)
    y = jnp.minimum(t2, y)
    m3 = jnp.maximum(t3, y)
    y = jnp.minimum(t3, y)
    m4 = jnp.maximum(t4, y)
    return (m1, m2, m3, m4)


@functools.partial(
    pl.kernel,
    out_type=jax.ShapeDtypeStruct((B, N), jnp.float32),
    mesh=_mesh,
    scratch_types=[
        pltpu.VMEM((N,), jnp.float32),
        pltpu.VMEM((N,), jnp.float32),
        pltpu.VMEM((N,), jnp.float32),
        pltpu.VMEM((CH,), jnp.float32),
        pltpu.SemaphoreType.DMA,
        pltpu.SemaphoreType.DMA,
        pltpu.SemaphoreType.DMA,
        pltpu.SemaphoreType.DMA,
        pltpu.SemaphoreType.DMA,
    ],
)
def _wta_sc(x_hbm, r_hbm, out_hbm, xa, xb, rbuf, r0buf, sem_x, sem_r0,
            sem_r1, sem_ra, sem_out):
    wid = lax.axis_index("s") * NC + lax.axis_index("c")
    neg = jnp.float32(-jnp.inf)

    dnums = lax.GatherDimensionNumbers(
        offset_dims=(), collapsed_slice_dims=(0,), start_index_map=(0,))

    def shuffle(v, idx):
        return lax.gather(v, idx.reshape(L, 1), dnums, slice_sizes=(1,),
                          mode=lax.GatherScatterMode.PROMISE_IN_BOUNDS)

    lane = lax.iota(jnp.int32, L)

    def allmax(v):
        for s in (8, 4, 2, 1):
            v = jnp.maximum(v, shuffle(v, lane ^ s))
        return v

    base = wid * ROWS_PER_W
    xbufs = [xa, xb]
    rsems = [sem_r0, sem_r1]

    h_x = pltpu.async_copy(x_hbm.at[base], xa, sem_x)
    h_r0 = pltpu.async_copy(r_hbm.at[base, pl.ds(0, CH)], r0buf, sem_ra)
    h_outs = []

    for row in range(ROWS_PER_W):
        ridx = base + row
        cur = xbufs[row % 2]
        nxt = xbufs[(row + 1) % 2]

        h_x.wait()

        h_r = pltpu.async_copy(
            r_hbm.at[ridx, pl.ds(CH, CH)], rbuf.at[pl.ds(CH, CH)], rsems[1])
        h_r0.wait()
        zero4 = (jnp.full((L,), neg, jnp.float32),) * TOPK
        sets = (zero4,) * NSETS
        for c in range(NCHUNK):
            if c > 0:
                h_r.wait()
            if 1 <= c < NCHUNK - 1:
                h_r = pltpu.async_copy(
                    r_hbm.at[ridx, pl.ds((c + 1) * CH, CH)],
                    rbuf.at[pl.ds((c + 1) * CH, CH)],
                    rsems[(c + 1) % 2])
            rsrc = r0buf if c == 0 else rbuf

            def p1_body(i, s, _c=c, _rsrc=rsrc):
                out = []
                for u in range(NSETS):
                    off = _c * CH + i + u * L
                    roff = i + u * L if _c == 0 else off
                    v = cur[pl.ds(off, L)] * _rsrc[pl.ds(roff, L)]
                    out.append(_insert4(s[u], v))
                return tuple(out)

            sets = plsc.parallel_loop(0, CH, step=NSETS * L, unroll=4,
                                      carry=sets)(p1_body)

        if row + 1 < ROWS_PER_W:
            for h in h_outs:
                h.wait()
            h_outs = []
            h_x = pltpu.async_copy(x_hbm.at[ridx + 1], nxt, sem_x)

        merged = sets[0]
        for u in range(1, NSETS):
            for v in sets[u]:
                merged = _insert4(merged, v)
        t1, t2, t3, t4 = merged

        thr = jnp.full((L,), neg, jnp.float32)
        for i in range(TOPK):
            m = allmax(jnp.maximum(jnp.maximum(t1, t2),
                                   jnp.maximum(t3, t4)))
            if i < TOPK - 1:
                t1 = jnp.where(t1 >= m, neg, t1)
                t2 = jnp.where(t2 >= m, neg, t2)
                t3 = jnp.where(t3 >= m, neg, t3)
                t4 = jnp.where(t4 >= m, neg, t4)
            thr = m

        for c in range(NCHUNK):
            rsrc = r0buf if c == 0 else rbuf

            def p2_body(i, _c=c, _rsrc=rsrc):
                off = _c * CH + i
                roff = i if _c == 0 else off
                xv = cur[pl.ds(off, L)]
                v = xv * _rsrc[pl.ds(roff, L)]
                cur[pl.ds(off, L)] = jnp.where(v >= thr, xv,
                                               jnp.float32(0.0))

            plsc.parallel_loop(0, CH, step=L, unroll=8)(p2_body)
            h_outs.append(pltpu.async_copy(
                cur.at[pl.ds(c * CH, CH)],
                out_hbm.at[ridx, pl.ds(c * CH, CH)], sem_out))
            if c == 0 and row + 1 < ROWS_PER_W:
                h_r0 = pltpu.async_copy(
                    r_hbm.at[ridx + 1, pl.ds(0, CH)], r0buf, sem_ra)

    for h in h_outs:
        h.wait()


import numpy as _np


def _threefry_uniform(seed, shape):
    rot_a = (13, 15, 26, 6)
    rot_b = (17, 29, 16, 24)
    k1 = _np.uint32(_np.int64(seed) >> 32)
    k2 = _np.uint32(_np.int64(seed) & 0xFFFFFFFF)
    ks = [k1, k2, _np.uint32(k1 ^ k2 ^ _np.uint32(0x1BD11BDA))]
    size = int(_np.prod(shape))
    idx = _np.arange(size, dtype=_np.uint64)
    x0 = (idx >> _np.uint64(32)).astype(_np.uint32)
    x1 = (idx & _np.uint64(0xFFFFFFFF)).astype(_np.uint32)
    x0 = (x0 + ks[0]).astype(_np.uint32)
    x1 = (x1 + ks[1]).astype(_np.uint32)
    for g in range(5):
        for r in rot_a if g % 2 == 0 else rot_b:
            x0 = (x0 + x1).astype(_np.uint32)
            x1 = (x1 << _np.uint32(r)) | (x1 >> _np.uint32(32 - r))
            x1 = x1 ^ x0
        x0 = (x0 + ks[(g + 1) % 3]).astype(_np.uint32)
        x1 = (x1 + ks[(g + 2) % 3] + _np.uint32(g + 1)).astype(_np.uint32)
    bits = x0 ^ x1
    f = (bits >> _np.uint32(9)) | _np.uint32(0x3F800000)
    f = f.view(_np.float32) - _np.float32(1.0)
    return _np.maximum(_np.float32(0.0), f).reshape(shape)


_R_NP = _threefry_uniform(42, (B, N))


def kernel(x):
    return _wta_sc(x, _R_NP)
